# baseline (speedup 1.0000x reference)
import functools

import jax
import jax.numpy as jnp
from jax import lax
from jax.experimental import pallas as pl
from jax.experimental.pallas import tpu as pltpu
from jax.experimental.pallas import tpu_sc as plsc

N = 10000
IN_DIM = 128
HID = 64
NPAD = 10240
EB = 128
E_TOT = 320000 + N
EPAD = 331776
BLK1 = 162
BLK2 = 81
RPT = NPAD // 16
ROWB = NPAD // 512

_f32 = jnp.float32


def _sc_mesh():
    return plsc.VectorSubcoreMesh(core_axis_name="c", subcore_axis_name="s")


def _gat_edge_pass(sid, gidx_hbm, sidx_hbm, hs_tab, ad_tab,
                   accH, z80, outH, bufs):
    r0 = sid * RPT
    pltpu.sync_copy(z80.at[pl.ds(r0, RPT), :], accH.at[pl.ds(r0, RPT), :])
    plsc.subcore_barrier()

    def prefetch(blk, B):
        gi, si, ssi, hr, br, ob, sg1, sg3, ss1 = B
        base = sid * (BLK1 * EB) + blk * EB
        pltpu.sync_copy(gidx_hbm.at[pl.ds(base, EB)], gi)
        pltpu.sync_copy(sidx_hbm.at[pl.ds(base, EB)], si)
        pltpu.async_copy(hs_tab.at[gi], hr, sg1)
        pltpu.async_copy(ad_tab.at[si], br, sg3)

    def wait_gathers(B):
        gi, si, ssi, hr, br, ob, sg1, sg3, ss1 = B
        pltpu.make_async_copy(hs_tab.at[gi], hr, sg1).wait()
        pltpu.make_async_copy(ad_tab.at[si], br, sg3).wait()

    def wait_scatters(B):
        gi, si, ssi, hr, br, ob, sg1, sg3, ss1 = B
        pltpu.make_async_copy(ob, accH.at[ssi], ss1).wait()

    def compute_and_scatter(blk, B):
        gi, si, ssi, hr, br, ob, sg1, sg3, ss1 = B
        base = sid * (BLK1 * EB) + blk * EB
        pltpu.sync_copy(sidx_hbm.at[pl.ds(base, EB)], ssi)

        lane_zero = jnp.zeros((16,), jnp.int32)
        lane_one = jnp.ones((16,), jnp.int32)

        @plsc.parallel_loop(0, EB, unroll=8)
        def edge(e):
            v = hr[e, pl.ds(64, 16)] + br[e, :]
            w = jnp.exp(jnp.maximum(v, 0.2 * v))
            ob[e, pl.ds(64, 16)] = w
            w0 = jnp.take_along_axis(w, lane_zero, axis=0)
            w1 = jnp.take_along_axis(w, lane_one, axis=0)
            ob[e, pl.ds(0, 16)] = hr[e, pl.ds(0, 16)] * w0
            ob[e, pl.ds(16, 16)] = hr[e, pl.ds(16, 16)] * w0
            ob[e, pl.ds(32, 16)] = hr[e, pl.ds(32, 16)] * w1
            ob[e, pl.ds(48, 16)] = hr[e, pl.ds(48, 16)] * w1

        pltpu.async_copy(ob, accH.at[ssi], ss1, add=True)

    B0, B1 = bufs
    npair = BLK1 // 2
    prefetch(0, B0)
    prefetch(1, B1)

    def pair(p, carry):
        wait_gathers(B0)

        @pl.when(p > 0)
        def _():
            wait_scatters(B0)

        compute_and_scatter(2 * p, B0)

        @pl.when(p < npair - 1)
        def _():
            prefetch(2 * p + 2, B0)

        wait_gathers(B1)

        @pl.when(p > 0)
        def _():
            wait_scatters(B1)

        compute_and_scatter(2 * p + 1, B1)

        @pl.when(p < npair - 1)
        def _():
            prefetch(2 * p + 3, B1)

        return carry

    lax.fori_loop(0, npair, pair, 0)
    wait_scatters(B0)
    wait_scatters(B1)
    plsc.subcore_barrier()
    pltpu.sync_copy(accH.at[pl.ds(r0, RPT), :], outH.at[pl.ds(r0, RPT), :])


def _buf_set():
    return [pltpu.VMEM((EB,), jnp.int32),
            pltpu.VMEM((EB,), jnp.int32),
            pltpu.VMEM((EB,), jnp.int32),
            pltpu.VMEM((EB, 80), _f32),
            pltpu.VMEM((EB, 16), _f32),
            pltpu.VMEM((EB, 80), _f32),
            pltpu.SemaphoreType.DMA,
            pltpu.SemaphoreType.DMA,
            pltpu.SemaphoreType.DMA]


def _sc_fu_body(sF, dF, hsF, adF, hsU, adU, z80,
                outF, outU, accH, *flat_bufs):
    cid = lax.axis_index("c")
    sid = lax.axis_index("s")
    nb = len(flat_bufs) // 2
    bufs = (flat_bufs[:nb], flat_bufs[nb:])

    @pl.when(cid == 0)
    def _():
        _gat_edge_pass(sid, sF, dF, hsF, adF, accH, z80, outF, bufs)

    @pl.when(cid == 1)
    def _():
        _gat_edge_pass(sid, dF, sF, hsU, adU, accH, z80, outU, bufs)


def _sc_fu_call(sF, dF, hsF, adF, hsU, adU, z80):
    out_type = [jax.ShapeDtypeStruct((NPAD, 80), _f32),
                jax.ShapeDtypeStruct((NPAD, 80), _f32)]
    scratch = ([pltpu.VMEM_SHARED((NPAD, 80), _f32)]
               + _buf_set() + _buf_set())
    fn = pl.kernel(_sc_fu_body, out_type=out_type, mesh=_sc_mesh(),
                   scratch_types=scratch,
                   compiler_params=pltpu.CompilerParams(
                       use_tc_tiling_on_sc=False))
    return fn(sF, dF, hsF, adF, hsU, adU, z80)


def _sc_l3_body(sF, dF, zas, ad3, z16, outW, accW, *flat_bufs):
    cid = lax.axis_index("c")
    sid = lax.axis_index("s")
    wid = cid * 16 + sid
    r0 = sid * RPT
    pltpu.sync_copy(z16.at[pl.ds(r0, RPT), :], accW.at[pl.ds(r0, RPT), :])
    plsc.subcore_barrier()
    lane0 = lax.iota(jnp.int32, 16) == 0
    lane1 = lax.iota(jnp.int32, 16) == 1
    nb = len(flat_bufs) // 2
    B0, B1 = flat_bufs[:nb], flat_bufs[nb:]

    def prefetch(blk, B):
        gi, si, ssi, zr, ar, wb, sg1, sg2, ss2 = B
        base = wid * (BLK2 * EB) + blk * EB
        pltpu.sync_copy(sF.at[pl.ds(base, EB)], gi)
        pltpu.sync_copy(dF.at[pl.ds(base, EB)], si)
        pltpu.async_copy(zas.at[gi], zr, sg1)
        pltpu.async_copy(ad3.at[si], ar, sg2)

    def wait_gathers(B):
        gi, si, ssi, zr, ar, wb, sg1, sg2, ss2 = B
        pltpu.make_async_copy(zas.at[gi], zr, sg1).wait()
        pltpu.make_async_copy(ad3.at[si], ar, sg2).wait()

    def wait_scatters(B):
        gi, si, ssi, zr, ar, wb, sg1, sg2, ss2 = B
        pltpu.make_async_copy(wb, accW.at[ssi], ss2).wait()

    def compute_and_scatter(blk, B):
        gi, si, ssi, zr, ar, wb, sg1, sg2, ss2 = B
        base = wid * (BLK2 * EB) + blk * EB
        pltpu.sync_copy(dF.at[pl.ds(base, EB)], ssi)

        lane_zero = jnp.zeros((16,), jnp.int32)
        lane_one = jnp.ones((16,), jnp.int32)

        @plsc.parallel_loop(0, EB, unroll=8)
        def edge(e):
            vz = zr[e, :]
            v = vz + ar[e, :]
            w = jnp.exp(jnp.maximum(v, 0.2 * v))
            w0 = jnp.take_along_axis(w, lane_zero, axis=0)
            zv = jnp.take_along_axis(vz, lane_one, axis=0)
            wb[e, :] = jnp.where(lane0, w0, jnp.where(lane1, w0 * zv, 0.0))

        pltpu.async_copy(wb, accW.at[ssi], ss2, add=True)

    npair = BLK2 // 2
    prefetch(0, B0)
    prefetch(1, B1)

    def pair(p, carry):
        wait_gathers(B0)

        @pl.when(p > 0)
        def _():
            wait_scatters(B0)

        compute_and_scatter(2 * p, B0)

        @pl.when(p < npair)
        def _():
            prefetch(2 * p + 2, B0)

        wait_gathers(B1)

        @pl.when(p > 0)
        def _():
            wait_scatters(B1)

        compute_and_scatter(2 * p + 1, B1)

        @pl.when(p < npair - 1)
        def _():
            prefetch(2 * p + 3, B1)

        return carry

    lax.fori_loop(0, npair, pair, 0)
    wait_gathers(B0)
    wait_scatters(B0)
    compute_and_scatter(BLK2 - 1, B0)
    wait_scatters(B0)
    wait_scatters(B1)
    plsc.subcore_barrier()
    orow = cid * NPAD + r0
    pltpu.sync_copy(accW.at[pl.ds(r0, RPT), :], outW.at[pl.ds(orow, RPT), :])


def _l3_buf_set():
    return [pltpu.VMEM((EB,), jnp.int32),
            pltpu.VMEM((EB,), jnp.int32),
            pltpu.VMEM((EB,), jnp.int32),
            pltpu.VMEM((EB, 16), _f32),
            pltpu.VMEM((EB, 16), _f32),
            pltpu.VMEM((EB, 16), _f32),
            pltpu.SemaphoreType.DMA,
            pltpu.SemaphoreType.DMA,
            pltpu.SemaphoreType.DMA]


def _sc_l3_call(sF, dF, zas, ad3, z16):
    out_type = jax.ShapeDtypeStruct((2 * NPAD, 16), _f32)
    scratch = ([pltpu.VMEM_SHARED((NPAD, 16), _f32)]
               + _l3_buf_set() + _l3_buf_set())
    fn = pl.kernel(_sc_l3_body, out_type=out_type, mesh=_sc_mesh(),
                   scratch_types=scratch,
                   compiler_params=pltpu.CompilerParams(
                       use_tc_tiling_on_sc=False))
    return fn(sF, dF, zas, ad3, z16)


def _tc_a_body(x_ref, Wf_ref, Wu_ref, PsF_ref, PdF_ref, PsU_ref, PdU_ref,
               hsF_ref, hsU_ref, adF_ref, adU_ref):
    x = x_ref[...]
    hf = jnp.dot(x, Wf_ref[...], preferred_element_type=_f32)
    hu = jnp.dot(x, Wu_ref[...], preferred_element_type=_f32)
    asf = jnp.dot(hf, PsF_ref[...], preferred_element_type=_f32)
    asu = jnp.dot(hu, PsU_ref[...], preferred_element_type=_f32)
    hsF_ref[...] = jnp.concatenate([hf, asf], axis=1)
    hsU_ref[...] = jnp.concatenate([hu, asu], axis=1)
    adF_ref[...] = jnp.dot(hf, PdF_ref[...], preferred_element_type=_f32)
    adU_ref[...] = jnp.dot(hu, PdU_ref[...], preferred_element_type=_f32)


def _tc_a_call(x_pad, Wf, Wu, PsF, PdF, PsU, PdU):
    w_spec = pl.BlockSpec((IN_DIM, 64), lambda i: (0, 0))
    p_spec = pl.BlockSpec((64, 16), lambda i: (0, 0))
    row80 = pl.BlockSpec((512, 80), lambda i: (i, 0))
    row16 = pl.BlockSpec((512, 16), lambda i: (i, 0))
    return pl.pallas_call(
        _tc_a_body,
        grid=(ROWB,),
        in_specs=[pl.BlockSpec((512, IN_DIM), lambda i: (i, 0)),
                  w_spec, w_spec, p_spec, p_spec, p_spec, p_spec],
        out_specs=[row80, row80, row16, row16],
        out_shape=[jax.ShapeDtypeStruct((NPAD, 80), _f32),
                   jax.ShapeDtypeStruct((NPAD, 80), _f32),
                   jax.ShapeDtypeStruct((NPAD, 16), _f32),
                   jax.ShapeDtypeStruct((NPAD, 16), _f32)],
    )(x_pad, Wf, Wu, PsF, PdF, PsU, PdU)


def _tc_b_body(aHF_ref, aHU_ref, bf_ref, bu_ref,
               Wfc_ref, bfc_ref, wo_ref, aos_ref, aod_ref,
               zas_ref, ad3_ref):
    eps = 1e-16
    hf = aHF_ref[...]
    f0 = hf[:, 0:32] / (hf[:, 64:65] + eps)
    f1 = hf[:, 32:64] / (hf[:, 65:66] + eps)
    outF = jnp.maximum(jnp.concatenate([f0, f1], axis=1) + bf_ref[...], 0.0)
    hu = aHU_ref[...]
    u0 = hu[:, 0:32] / (hu[:, 64:65] + eps)
    u1 = hu[:, 32:64] / (hu[:, 65:66] + eps)
    outU = jnp.maximum(jnp.concatenate([u0, u1], axis=1) + bu_ref[...], 0.0)
    cat = jnp.concatenate([outF, outU], axis=1)
    h2 = jnp.dot(cat, Wfc_ref[...], preferred_element_type=_f32) + bfc_ref[...]
    h2 = jnp.maximum(h2, 0.0)
    z = jnp.sum(h2 * wo_ref[...], axis=1, keepdims=True)
    zeros14 = jnp.zeros((z.shape[0], 14), _f32)
    zeros15 = jnp.zeros((z.shape[0], 15), _f32)
    zas_ref[...] = jnp.concatenate([z * aos_ref[0, 0], z, zeros14], axis=1)
    ad3_ref[...] = jnp.concatenate([z * aod_ref[0, 0], zeros15], axis=1)


def _tc_b_call(aHF, aHU, bf, bu, Wfc, bfc, wo, aos, aod):
    row80 = pl.BlockSpec((512, 80), lambda i: (i, 0))
    row16 = pl.BlockSpec((512, 16), lambda i: (i, 0))
    b_spec = pl.BlockSpec((1, 64), lambda i: (0, 0))
    s_spec = pl.BlockSpec((1, 1), lambda i: (0, 0))
    return pl.pallas_call(
        _tc_b_body,
        grid=(ROWB,),
        in_specs=[row80, row80, b_spec, b_spec,
                  pl.BlockSpec((128, 64), lambda i: (0, 0)), b_spec,
                  b_spec, s_spec, s_spec],
        out_specs=[row16, row16],
        out_shape=[jax.ShapeDtypeStruct((NPAD, 16), _f32),
                   jax.ShapeDtypeStruct((NPAD, 16), _f32)],
    )(aHF, aHU, bf, bu, Wfc, bfc, wo, aos, aod)


def _tc_c_body(pW0_ref, pW1_ref, bo_ref, out_ref):
    den = pW0_ref[:, 0:1] + pW1_ref[:, 0:1]
    num = pW0_ref[:, 1:2] + pW1_ref[:, 1:2]
    out_ref[...] = jax.nn.sigmoid(num / (den + 1e-16) + bo_ref[0, 0])


def _tc_c_call(pW, bo2):
    lo = pl.BlockSpec((512, 16), lambda i: (i, 0))
    hi = pl.BlockSpec((512, 16), lambda i: (i + ROWB, 0))
    s_spec = pl.BlockSpec((1, 1), lambda i: (0, 0))
    return pl.pallas_call(
        _tc_c_body,
        grid=(ROWB,),
        in_specs=[lo, hi, s_spec],
        out_specs=pl.BlockSpec((512, 1), lambda i: (i, 0)),
        out_shape=jax.ShapeDtypeStruct((NPAD, 1), _f32),
    )(pW, pW, bo2)


def _alpha_proj(a):
    P = jnp.zeros((64, 16), _f32)
    P = P.at[0:32, 0].set(a[0])
    P = P.at[32:64, 1].set(a[1])
    return P


def kernel(x, edge_index, Wf, af_src, af_dst, bf, Wu, au_src, au_dst, bu,
           Wfc, bfc, Wo, ao_src, ao_dst, bo):
    src = edge_index[0]
    dst = edge_index[1]
    loop = jnp.arange(N, dtype=jnp.int32)
    padv = jnp.full((EPAD - E_TOT,), N, dtype=jnp.int32)
    sF = jnp.concatenate([src, loop, padv])
    dF = jnp.concatenate([dst, loop, padv])
    x_pad = jnp.zeros((NPAD, IN_DIM), _f32).at[:N].set(x)
    z80 = jnp.zeros((NPAD, 80), _f32)
    z16 = jnp.zeros((NPAD, 16), _f32)

    hsF, hsU, adF, adU = _tc_a_call(
        x_pad, Wf, Wu, _alpha_proj(af_src), _alpha_proj(af_dst),
        _alpha_proj(au_src), _alpha_proj(au_dst))

    outF, outU = _sc_fu_call(sF, dF, hsF, adF, hsU, adU, z80)

    zas, ad3 = _tc_b_call(outF, outU,
                          bf.reshape(1, 64), bu.reshape(1, 64),
                          Wfc, bfc.reshape(1, 64), Wo.reshape(1, 64),
                          ao_src.reshape(1, 1), ao_dst.reshape(1, 1))

    pW = _sc_l3_call(sF, dF, zas, ad3, z16)

    out = _tc_c_call(pW, bo.reshape(1, 1))
    return out[:N]


# speedup vs baseline: 1.4280x; 1.4280x over previous
import functools

import jax
import jax.numpy as jnp
from jax import lax
from jax.experimental import pallas as pl
from jax.experimental.pallas import tpu as pltpu
from jax.experimental.pallas import tpu_sc as plsc

N = 10000
IN_DIM = 128
HID = 64
NPAD = 10240
EB = 128
E_TOT = 320000 + N
EPAD = 331776
BLK1 = 162
BLK2 = 81
RPT = NPAD // 16
ROWB = NPAD // 512

_f32 = jnp.float32


def _sc_mesh():
    return plsc.VectorSubcoreMesh(core_axis_name="c", subcore_axis_name="s")


def _gat_edge_pass(sid, gidx_hbm, sidx_hbm, hs_tab, ad_tab,
                   accH, z80, outH, bufs):
    r0 = sid * RPT
    pltpu.sync_copy(z80.at[pl.ds(r0, RPT), :], accH.at[pl.ds(r0, RPT), :])
    plsc.subcore_barrier()

    def prefetch_idx(blk, B):
        gi, si, ssi, hr, br, ob, sg1, sg3, ss1, sx1, sx2, sx3 = B
        base = sid * (BLK1 * EB) + blk * EB
        pltpu.async_copy(gidx_hbm.at[pl.ds(base, EB)], gi, sx1)
        pltpu.async_copy(sidx_hbm.at[pl.ds(base, EB)], si, sx2)

    def issue_gathers(blk, B):
        gi, si, ssi, hr, br, ob, sg1, sg3, ss1, sx1, sx2, sx3 = B
        base = sid * (BLK1 * EB) + blk * EB
        pltpu.make_async_copy(gidx_hbm.at[pl.ds(base, EB)], gi, sx1).wait()
        pltpu.make_async_copy(sidx_hbm.at[pl.ds(base, EB)], si, sx2).wait()
        pltpu.async_copy(hs_tab.at[gi], hr, sg1)
        pltpu.async_copy(ad_tab.at[si], br, sg3)

    def wait_gathers(B):
        gi, si, ssi, hr, br, ob, sg1, sg3, ss1, sx1, sx2, sx3 = B
        pltpu.make_async_copy(hs_tab.at[gi], hr, sg1).wait()
        pltpu.make_async_copy(ad_tab.at[si], br, sg3).wait()

    def wait_scatters(B):
        gi, si, ssi, hr, br, ob, sg1, sg3, ss1, sx1, sx2, sx3 = B
        pltpu.make_async_copy(ob, accH.at[ssi], ss1).wait()

    def compute_and_scatter(blk, B):
        gi, si, ssi, hr, br, ob, sg1, sg3, ss1, sx1, sx2, sx3 = B
        base = sid * (BLK1 * EB) + blk * EB
        pltpu.async_copy(sidx_hbm.at[pl.ds(base, EB)], ssi, sx3)

        lane_zero = jnp.zeros((16,), jnp.int32)
        lane_one = jnp.ones((16,), jnp.int32)

        @plsc.parallel_loop(0, EB, unroll=8)
        def edge(e):
            v = hr[e, pl.ds(64, 16)] + br[e, :]
            w = jnp.exp(jnp.maximum(v, 0.2 * v))
            ob[e, pl.ds(64, 16)] = w
            w0 = jnp.take_along_axis(w, lane_zero, axis=0)
            w1 = jnp.take_along_axis(w, lane_one, axis=0)
            ob[e, pl.ds(0, 16)] = hr[e, pl.ds(0, 16)] * w0
            ob[e, pl.ds(16, 16)] = hr[e, pl.ds(16, 16)] * w0
            ob[e, pl.ds(32, 16)] = hr[e, pl.ds(32, 16)] * w1
            ob[e, pl.ds(48, 16)] = hr[e, pl.ds(48, 16)] * w1

        pltpu.make_async_copy(sidx_hbm.at[pl.ds(base, EB)], ssi, sx3).wait()
        pltpu.async_copy(ob, accH.at[ssi], ss1, add=True)

    B0, B1 = bufs
    npair = BLK1 // 2
    prefetch_idx(0, B0)
    prefetch_idx(1, B1)
    issue_gathers(0, B0)
    issue_gathers(1, B1)

    def pair(p, carry):
        wait_gathers(B0)

        @pl.when(p < npair - 1)
        def _():
            prefetch_idx(2 * p + 2, B0)

        @pl.when(p > 0)
        def _():
            wait_scatters(B0)

        compute_and_scatter(2 * p, B0)

        @pl.when(p < npair - 1)
        def _():
            issue_gathers(2 * p + 2, B0)

        wait_gathers(B1)

        @pl.when(p < npair - 1)
        def _():
            prefetch_idx(2 * p + 3, B1)

        @pl.when(p > 0)
        def _():
            wait_scatters(B1)

        compute_and_scatter(2 * p + 1, B1)

        @pl.when(p < npair - 1)
        def _():
            issue_gathers(2 * p + 3, B1)

        return carry

    lax.fori_loop(0, npair, pair, 0)
    wait_scatters(B0)
    wait_scatters(B1)
    plsc.subcore_barrier()
    pltpu.sync_copy(accH.at[pl.ds(r0, RPT), :], outH.at[pl.ds(r0, RPT), :])


def _buf_set():
    return [pltpu.VMEM((EB,), jnp.int32),
            pltpu.VMEM((EB,), jnp.int32),
            pltpu.VMEM((EB,), jnp.int32),
            pltpu.VMEM((EB, 80), _f32),
            pltpu.VMEM((EB, 16), _f32),
            pltpu.VMEM((EB, 80), _f32),
            pltpu.SemaphoreType.DMA,
            pltpu.SemaphoreType.DMA,
            pltpu.SemaphoreType.DMA,
            pltpu.SemaphoreType.DMA,
            pltpu.SemaphoreType.DMA,
            pltpu.SemaphoreType.DMA]


def _sc_fu_body(sF, dF, hsF, adF, hsU, adU, z80,
                outF, outU, accH, *flat_bufs):
    cid = lax.axis_index("c")
    sid = lax.axis_index("s")
    nb = len(flat_bufs) // 2
    bufs = (flat_bufs[:nb], flat_bufs[nb:])

    @pl.when(cid == 0)
    def _():
        _gat_edge_pass(sid, sF, dF, hsF, adF, accH, z80, outF, bufs)

    @pl.when(cid == 1)
    def _():
        _gat_edge_pass(sid, dF, sF, hsU, adU, accH, z80, outU, bufs)


def _sc_fu_call(sF, dF, hsF, adF, hsU, adU, z80):
    out_type = [jax.ShapeDtypeStruct((NPAD, 80), _f32),
                jax.ShapeDtypeStruct((NPAD, 80), _f32)]
    scratch = ([pltpu.VMEM_SHARED((NPAD, 80), _f32)]
               + _buf_set() + _buf_set())
    fn = pl.kernel(_sc_fu_body, out_type=out_type, mesh=_sc_mesh(),
                   scratch_types=scratch,
                   compiler_params=pltpu.CompilerParams(
                       use_tc_tiling_on_sc=False))
    return fn(sF, dF, hsF, adF, hsU, adU, z80)


def _sc_l3_body(sF, dF, zas, ad3, z16, outW, accW, *flat_bufs):
    cid = lax.axis_index("c")
    sid = lax.axis_index("s")
    wid = cid * 16 + sid
    r0 = sid * RPT
    pltpu.sync_copy(z16.at[pl.ds(r0, RPT), :], accW.at[pl.ds(r0, RPT), :])
    plsc.subcore_barrier()
    lane0 = lax.iota(jnp.int32, 16) == 0
    lane1 = lax.iota(jnp.int32, 16) == 1
    nb = len(flat_bufs) // 2
    B0, B1 = flat_bufs[:nb], flat_bufs[nb:]

    def prefetch_idx(blk, B):
        gi, si, ssi, zr, ar, wb, sg1, sg2, ss2, sx1, sx2, sx3 = B
        base = wid * (BLK2 * EB) + blk * EB
        pltpu.async_copy(sF.at[pl.ds(base, EB)], gi, sx1)
        pltpu.async_copy(dF.at[pl.ds(base, EB)], si, sx2)

    def issue_gathers(blk, B):
        gi, si, ssi, zr, ar, wb, sg1, sg2, ss2, sx1, sx2, sx3 = B
        base = wid * (BLK2 * EB) + blk * EB
        pltpu.make_async_copy(sF.at[pl.ds(base, EB)], gi, sx1).wait()
        pltpu.make_async_copy(dF.at[pl.ds(base, EB)], si, sx2).wait()
        pltpu.async_copy(zas.at[gi], zr, sg1)
        pltpu.async_copy(ad3.at[si], ar, sg2)

    def wait_gathers(B):
        gi, si, ssi, zr, ar, wb, sg1, sg2, ss2, sx1, sx2, sx3 = B
        pltpu.make_async_copy(zas.at[gi], zr, sg1).wait()
        pltpu.make_async_copy(ad3.at[si], ar, sg2).wait()

    def wait_scatters(B):
        gi, si, ssi, zr, ar, wb, sg1, sg2, ss2, sx1, sx2, sx3 = B
        pltpu.make_async_copy(wb, accW.at[ssi], ss2).wait()

    def compute_and_scatter(blk, B):
        gi, si, ssi, zr, ar, wb, sg1, sg2, ss2, sx1, sx2, sx3 = B
        base = wid * (BLK2 * EB) + blk * EB
        pltpu.async_copy(dF.at[pl.ds(base, EB)], ssi, sx3)

        lane_zero = jnp.zeros((16,), jnp.int32)
        lane_one = jnp.ones((16,), jnp.int32)

        @plsc.parallel_loop(0, EB, unroll=8)
        def edge(e):
            vz = zr[e, :]
            v = vz + ar[e, :]
            w = jnp.exp(jnp.maximum(v, 0.2 * v))
            w0 = jnp.take_along_axis(w, lane_zero, axis=0)
            zv = jnp.take_along_axis(vz, lane_one, axis=0)
            wb[e, :] = jnp.where(lane0, w0, jnp.where(lane1, w0 * zv, 0.0))

        pltpu.make_async_copy(dF.at[pl.ds(base, EB)], ssi, sx3).wait()
        pltpu.async_copy(wb, accW.at[ssi], ss2, add=True)

    npair = BLK2 // 2
    prefetch_idx(0, B0)
    prefetch_idx(1, B1)
    issue_gathers(0, B0)
    issue_gathers(1, B1)

    def pair(p, carry):
        wait_gathers(B0)

        @pl.when(p < npair)
        def _():
            prefetch_idx(2 * p + 2, B0)

        @pl.when(p > 0)
        def _():
            wait_scatters(B0)

        compute_and_scatter(2 * p, B0)

        @pl.when(p < npair)
        def _():
            issue_gathers(2 * p + 2, B0)

        wait_gathers(B1)

        @pl.when(p < npair - 1)
        def _():
            prefetch_idx(2 * p + 3, B1)

        @pl.when(p > 0)
        def _():
            wait_scatters(B1)

        compute_and_scatter(2 * p + 1, B1)

        @pl.when(p < npair - 1)
        def _():
            issue_gathers(2 * p + 3, B1)

        return carry

    lax.fori_loop(0, npair, pair, 0)
    wait_gathers(B0)
    wait_scatters(B0)
    compute_and_scatter(BLK2 - 1, B0)
    wait_scatters(B0)
    wait_scatters(B1)
    plsc.subcore_barrier()
    orow = cid * NPAD + r0
    pltpu.sync_copy(accW.at[pl.ds(r0, RPT), :], outW.at[pl.ds(orow, RPT), :])


def _l3_buf_set():
    return [pltpu.VMEM((EB,), jnp.int32),
            pltpu.VMEM((EB,), jnp.int32),
            pltpu.VMEM((EB,), jnp.int32),
            pltpu.VMEM((EB, 16), _f32),
            pltpu.VMEM((EB, 16), _f32),
            pltpu.VMEM((EB, 16), _f32),
            pltpu.SemaphoreType.DMA,
            pltpu.SemaphoreType.DMA,
            pltpu.SemaphoreType.DMA,
            pltpu.SemaphoreType.DMA,
            pltpu.SemaphoreType.DMA,
            pltpu.SemaphoreType.DMA]


def _sc_l3_call(sF, dF, zas, ad3, z16):
    out_type = jax.ShapeDtypeStruct((2 * NPAD, 16), _f32)
    scratch = ([pltpu.VMEM_SHARED((NPAD, 16), _f32)]
               + _l3_buf_set() + _l3_buf_set())
    fn = pl.kernel(_sc_l3_body, out_type=out_type, mesh=_sc_mesh(),
                   scratch_types=scratch,
                   compiler_params=pltpu.CompilerParams(
                       use_tc_tiling_on_sc=False))
    return fn(sF, dF, zas, ad3, z16)


def _tc_a_body(x_ref, Wf_ref, Wu_ref, PsF_ref, PdF_ref, PsU_ref, PdU_ref,
               hsF_ref, hsU_ref, adF_ref, adU_ref):
    x = x_ref[...]
    hf = jnp.dot(x, Wf_ref[...], preferred_element_type=_f32)
    hu = jnp.dot(x, Wu_ref[...], preferred_element_type=_f32)
    asf = jnp.dot(hf, PsF_ref[...], preferred_element_type=_f32)
    asu = jnp.dot(hu, PsU_ref[...], preferred_element_type=_f32)
    hsF_ref[...] = jnp.concatenate([hf, asf], axis=1)
    hsU_ref[...] = jnp.concatenate([hu, asu], axis=1)
    adF_ref[...] = jnp.dot(hf, PdF_ref[...], preferred_element_type=_f32)
    adU_ref[...] = jnp.dot(hu, PdU_ref[...], preferred_element_type=_f32)


def _tc_a_call(x_pad, Wf, Wu, PsF, PdF, PsU, PdU):
    w_spec = pl.BlockSpec((IN_DIM, 64), lambda i: (0, 0))
    p_spec = pl.BlockSpec((64, 16), lambda i: (0, 0))
    row80 = pl.BlockSpec((512, 80), lambda i: (i, 0))
    row16 = pl.BlockSpec((512, 16), lambda i: (i, 0))
    return pl.pallas_call(
        _tc_a_body,
        grid=(ROWB,),
        in_specs=[pl.BlockSpec((512, IN_DIM), lambda i: (i, 0)),
                  w_spec, w_spec, p_spec, p_spec, p_spec, p_spec],
        out_specs=[row80, row80, row16, row16],
        out_shape=[jax.ShapeDtypeStruct((NPAD, 80), _f32),
                   jax.ShapeDtypeStruct((NPAD, 80), _f32),
                   jax.ShapeDtypeStruct((NPAD, 16), _f32),
                   jax.ShapeDtypeStruct((NPAD, 16), _f32)],
    )(x_pad, Wf, Wu, PsF, PdF, PsU, PdU)


def _tc_b_body(aHF_ref, aHU_ref, bf_ref, bu_ref,
               Wfc_ref, bfc_ref, wo_ref, aos_ref, aod_ref,
               zas_ref, ad3_ref):
    eps = 1e-16
    hf = aHF_ref[...]
    f0 = hf[:, 0:32] / (hf[:, 64:65] + eps)
    f1 = hf[:, 32:64] / (hf[:, 65:66] + eps)
    outF = jnp.maximum(jnp.concatenate([f0, f1], axis=1) + bf_ref[...], 0.0)
    hu = aHU_ref[...]
    u0 = hu[:, 0:32] / (hu[:, 64:65] + eps)
    u1 = hu[:, 32:64] / (hu[:, 65:66] + eps)
    outU = jnp.maximum(jnp.concatenate([u0, u1], axis=1) + bu_ref[...], 0.0)
    cat = jnp.concatenate([outF, outU], axis=1)
    h2 = jnp.dot(cat, Wfc_ref[...], preferred_element_type=_f32) + bfc_ref[...]
    h2 = jnp.maximum(h2, 0.0)
    z = jnp.sum(h2 * wo_ref[...], axis=1, keepdims=True)
    zeros14 = jnp.zeros((z.shape[0], 14), _f32)
    zeros15 = jnp.zeros((z.shape[0], 15), _f32)
    zas_ref[...] = jnp.concatenate([z * aos_ref[0, 0], z, zeros14], axis=1)
    ad3_ref[...] = jnp.concatenate([z * aod_ref[0, 0], zeros15], axis=1)


def _tc_b_call(aHF, aHU, bf, bu, Wfc, bfc, wo, aos, aod):
    row80 = pl.BlockSpec((512, 80), lambda i: (i, 0))
    row16 = pl.BlockSpec((512, 16), lambda i: (i, 0))
    b_spec = pl.BlockSpec((1, 64), lambda i: (0, 0))
    s_spec = pl.BlockSpec((1, 1), lambda i: (0, 0))
    return pl.pallas_call(
        _tc_b_body,
        grid=(ROWB,),
        in_specs=[row80, row80, b_spec, b_spec,
                  pl.BlockSpec((128, 64), lambda i: (0, 0)), b_spec,
                  b_spec, s_spec, s_spec],
        out_specs=[row16, row16],
        out_shape=[jax.ShapeDtypeStruct((NPAD, 16), _f32),
                   jax.ShapeDtypeStruct((NPAD, 16), _f32)],
    )(aHF, aHU, bf, bu, Wfc, bfc, wo, aos, aod)


def _tc_c_body(pW0_ref, pW1_ref, bo_ref, out_ref):
    den = pW0_ref[:, 0:1] + pW1_ref[:, 0:1]
    num = pW0_ref[:, 1:2] + pW1_ref[:, 1:2]
    out_ref[...] = jax.nn.sigmoid(num / (den + 1e-16) + bo_ref[0, 0])


def _tc_c_call(pW, bo2):
    lo = pl.BlockSpec((512, 16), lambda i: (i, 0))
    hi = pl.BlockSpec((512, 16), lambda i: (i + ROWB, 0))
    s_spec = pl.BlockSpec((1, 1), lambda i: (0, 0))
    return pl.pallas_call(
        _tc_c_body,
        grid=(ROWB,),
        in_specs=[lo, hi, s_spec],
        out_specs=pl.BlockSpec((512, 1), lambda i: (i, 0)),
        out_shape=jax.ShapeDtypeStruct((NPAD, 1), _f32),
    )(pW, pW, bo2)


def _alpha_proj(a):
    P = jnp.zeros((64, 16), _f32)
    P = P.at[0:32, 0].set(a[0])
    P = P.at[32:64, 1].set(a[1])
    return P


def kernel(x, edge_index, Wf, af_src, af_dst, bf, Wu, au_src, au_dst, bu,
           Wfc, bfc, Wo, ao_src, ao_dst, bo):
    src = edge_index[0]
    dst = edge_index[1]
    loop = jnp.arange(N, dtype=jnp.int32)
    padv = jnp.full((EPAD - E_TOT,), N, dtype=jnp.int32)
    sF = jnp.concatenate([src, loop, padv])
    dF = jnp.concatenate([dst, loop, padv])
    x_pad = jnp.zeros((NPAD, IN_DIM), _f32).at[:N].set(x)
    z80 = jnp.zeros((NPAD, 80), _f32)
    z16 = jnp.zeros((NPAD, 16), _f32)

    hsF, hsU, adF, adU = _tc_a_call(
        x_pad, Wf, Wu, _alpha_proj(af_src), _alpha_proj(af_dst),
        _alpha_proj(au_src), _alpha_proj(au_dst))

    outF, outU = _sc_fu_call(sF, dF, hsF, adF, hsU, adU, z80)

    zas, ad3 = _tc_b_call(outF, outU,
                          bf.reshape(1, 64), bu.reshape(1, 64),
                          Wfc, bfc.reshape(1, 64), Wo.reshape(1, 64),
                          ao_src.reshape(1, 1), ao_dst.reshape(1, 1))

    pW = _sc_l3_call(sF, dF, zas, ad3, z16)

    out = _tc_c_call(pW, bo.reshape(1, 1))
    return out[:N]


# speedup vs baseline: 1.4296x; 1.0011x over previous
import functools

import jax
import jax.numpy as jnp
from jax import lax
from jax.experimental import pallas as pl
from jax.experimental.pallas import tpu as pltpu
from jax.experimental.pallas import tpu_sc as plsc

N = 10000
IN_DIM = 128
HID = 64
NPAD = 10240
EB = 128
E_TOT = 320000 + N
EPAD = 331776
BLK1 = 162
BLK2 = 81
RPT = NPAD // 16
ROWB = NPAD // 512

_f32 = jnp.float32


def _sc_mesh():
    return plsc.VectorSubcoreMesh(core_axis_name="c", subcore_axis_name="s")


def _gat_edge_pass(sid, gidx_hbm, sidx_hbm, hs_tab, ad_tab,
                   accH, z80, outH, bufs):
    r0 = sid * RPT
    pltpu.sync_copy(z80.at[pl.ds(r0, RPT), :], accH.at[pl.ds(r0, RPT), :])
    plsc.subcore_barrier()

    def prefetch_idx(blk, B):
        gi, si, ssi, hr, br, ob, sg1, sg3, ss1, sx1, sx2, sx3 = B
        base = sid * (BLK1 * EB) + blk * EB
        pltpu.async_copy(gidx_hbm.at[pl.ds(base, EB)], gi, sx1)
        pltpu.async_copy(sidx_hbm.at[pl.ds(base, EB)], si, sx2)

    def issue_gathers(blk, B):
        gi, si, ssi, hr, br, ob, sg1, sg3, ss1, sx1, sx2, sx3 = B
        base = sid * (BLK1 * EB) + blk * EB
        pltpu.make_async_copy(gidx_hbm.at[pl.ds(base, EB)], gi, sx1).wait()
        pltpu.make_async_copy(sidx_hbm.at[pl.ds(base, EB)], si, sx2).wait()
        pltpu.async_copy(hs_tab.at[gi], hr, sg1)
        pltpu.async_copy(ad_tab.at[si], br, sg3)

    def wait_gathers(B):
        gi, si, ssi, hr, br, ob, sg1, sg3, ss1, sx1, sx2, sx3 = B
        pltpu.make_async_copy(hs_tab.at[gi], hr, sg1).wait()
        pltpu.make_async_copy(ad_tab.at[si], br, sg3).wait()

    def wait_scatters(B):
        gi, si, ssi, hr, br, ob, sg1, sg3, ss1, sx1, sx2, sx3 = B
        pltpu.make_async_copy(ob, accH.at[ssi], ss1).wait()

    def compute_and_scatter(blk, B):
        gi, si, ssi, hr, br, ob, sg1, sg3, ss1, sx1, sx2, sx3 = B
        base = sid * (BLK1 * EB) + blk * EB
        pltpu.async_copy(sidx_hbm.at[pl.ds(base, EB)], ssi, sx3)

        lane_zero = jnp.zeros((16,), jnp.int32)
        lane_one = jnp.ones((16,), jnp.int32)

        @plsc.parallel_loop(0, EB, unroll=16)
        def edge(e):
            v = hr[e, pl.ds(64, 16)] + br[e, :]
            w = jnp.exp(jnp.maximum(v, 0.2 * v))
            ob[e, pl.ds(64, 16)] = w
            w0 = jnp.take_along_axis(w, lane_zero, axis=0)
            w1 = jnp.take_along_axis(w, lane_one, axis=0)
            ob[e, pl.ds(0, 16)] = hr[e, pl.ds(0, 16)] * w0
            ob[e, pl.ds(16, 16)] = hr[e, pl.ds(16, 16)] * w0
            ob[e, pl.ds(32, 16)] = hr[e, pl.ds(32, 16)] * w1
            ob[e, pl.ds(48, 16)] = hr[e, pl.ds(48, 16)] * w1

        pltpu.make_async_copy(sidx_hbm.at[pl.ds(base, EB)], ssi, sx3).wait()
        pltpu.async_copy(ob, accH.at[ssi], ss1, add=True)

    B0, B1 = bufs
    npair = BLK1 // 2
    prefetch_idx(0, B0)
    prefetch_idx(1, B1)
    issue_gathers(0, B0)
    issue_gathers(1, B1)

    def pair(p, carry):
        wait_gathers(B0)

        @pl.when(p < npair - 1)
        def _():
            prefetch_idx(2 * p + 2, B0)

        @pl.when(p > 0)
        def _():
            wait_scatters(B0)

        compute_and_scatter(2 * p, B0)

        @pl.when(p < npair - 1)
        def _():
            issue_gathers(2 * p + 2, B0)

        wait_gathers(B1)

        @pl.when(p < npair - 1)
        def _():
            prefetch_idx(2 * p + 3, B1)

        @pl.when(p > 0)
        def _():
            wait_scatters(B1)

        compute_and_scatter(2 * p + 1, B1)

        @pl.when(p < npair - 1)
        def _():
            issue_gathers(2 * p + 3, B1)

        return carry

    lax.fori_loop(0, npair, pair, 0)
    wait_scatters(B0)
    wait_scatters(B1)
    plsc.subcore_barrier()
    pltpu.sync_copy(accH.at[pl.ds(r0, RPT), :], outH.at[pl.ds(r0, RPT), :])


def _buf_set():
    return [pltpu.VMEM((EB,), jnp.int32),
            pltpu.VMEM((EB,), jnp.int32),
            pltpu.VMEM((EB,), jnp.int32),
            pltpu.VMEM((EB, 80), _f32),
            pltpu.VMEM((EB, 16), _f32),
            pltpu.VMEM((EB, 80), _f32),
            pltpu.SemaphoreType.DMA,
            pltpu.SemaphoreType.DMA,
            pltpu.SemaphoreType.DMA,
            pltpu.SemaphoreType.DMA,
            pltpu.SemaphoreType.DMA,
            pltpu.SemaphoreType.DMA]


def _sc_fu_body(sF, dF, hsF, adF, hsU, adU, z80,
                outF, outU, accH, *flat_bufs):
    cid = lax.axis_index("c")
    sid = lax.axis_index("s")
    nb = len(flat_bufs) // 2
    bufs = (flat_bufs[:nb], flat_bufs[nb:])

    @pl.when(cid == 0)
    def _():
        _gat_edge_pass(sid, sF, dF, hsF, adF, accH, z80, outF, bufs)

    @pl.when(cid == 1)
    def _():
        _gat_edge_pass(sid, dF, sF, hsU, adU, accH, z80, outU, bufs)


def _sc_fu_call(sF, dF, hsF, adF, hsU, adU, z80):
    out_type = [jax.ShapeDtypeStruct((NPAD, 80), _f32),
                jax.ShapeDtypeStruct((NPAD, 80), _f32)]
    scratch = ([pltpu.VMEM_SHARED((NPAD, 80), _f32)]
               + _buf_set() + _buf_set())
    fn = pl.kernel(_sc_fu_body, out_type=out_type, mesh=_sc_mesh(),
                   scratch_types=scratch,
                   compiler_params=pltpu.CompilerParams(
                       use_tc_tiling_on_sc=False))
    return fn(sF, dF, hsF, adF, hsU, adU, z80)


def _sc_l3_body(sF, dF, zas, ad3, z16, outW, accW, *flat_bufs):
    cid = lax.axis_index("c")
    sid = lax.axis_index("s")
    wid = cid * 16 + sid
    r0 = sid * RPT
    pltpu.sync_copy(z16.at[pl.ds(r0, RPT), :], accW.at[pl.ds(r0, RPT), :])
    plsc.subcore_barrier()
    lane0 = lax.iota(jnp.int32, 16) == 0
    lane1 = lax.iota(jnp.int32, 16) == 1
    nb = len(flat_bufs) // 2
    B0, B1 = flat_bufs[:nb], flat_bufs[nb:]

    def prefetch_idx(blk, B):
        gi, si, ssi, zr, ar, wb, sg1, sg2, ss2, sx1, sx2, sx3 = B
        base = wid * (BLK2 * EB) + blk * EB
        pltpu.async_copy(sF.at[pl.ds(base, EB)], gi, sx1)
        pltpu.async_copy(dF.at[pl.ds(base, EB)], si, sx2)

    def issue_gathers(blk, B):
        gi, si, ssi, zr, ar, wb, sg1, sg2, ss2, sx1, sx2, sx3 = B
        base = wid * (BLK2 * EB) + blk * EB
        pltpu.make_async_copy(sF.at[pl.ds(base, EB)], gi, sx1).wait()
        pltpu.make_async_copy(dF.at[pl.ds(base, EB)], si, sx2).wait()
        pltpu.async_copy(zas.at[gi], zr, sg1)
        pltpu.async_copy(ad3.at[si], ar, sg2)

    def wait_gathers(B):
        gi, si, ssi, zr, ar, wb, sg1, sg2, ss2, sx1, sx2, sx3 = B
        pltpu.make_async_copy(zas.at[gi], zr, sg1).wait()
        pltpu.make_async_copy(ad3.at[si], ar, sg2).wait()

    def wait_scatters(B):
        gi, si, ssi, zr, ar, wb, sg1, sg2, ss2, sx1, sx2, sx3 = B
        pltpu.make_async_copy(wb, accW.at[ssi], ss2).wait()

    def compute_and_scatter(blk, B):
        gi, si, ssi, zr, ar, wb, sg1, sg2, ss2, sx1, sx2, sx3 = B
        base = wid * (BLK2 * EB) + blk * EB
        pltpu.async_copy(dF.at[pl.ds(base, EB)], ssi, sx3)

        lane_zero = jnp.zeros((16,), jnp.int32)
        lane_one = jnp.ones((16,), jnp.int32)

        @plsc.parallel_loop(0, EB, unroll=16)
        def edge(e):
            vz = zr[e, :]
            v = vz + ar[e, :]
            w = jnp.exp(jnp.maximum(v, 0.2 * v))
            w0 = jnp.take_along_axis(w, lane_zero, axis=0)
            zv = jnp.take_along_axis(vz, lane_one, axis=0)
            wb[e, :] = jnp.where(lane0, w0, jnp.where(lane1, w0 * zv, 0.0))

        pltpu.make_async_copy(dF.at[pl.ds(base, EB)], ssi, sx3).wait()
        pltpu.async_copy(wb, accW.at[ssi], ss2, add=True)

    npair = BLK2 // 2
    prefetch_idx(0, B0)
    prefetch_idx(1, B1)
    issue_gathers(0, B0)
    issue_gathers(1, B1)

    def pair(p, carry):
        wait_gathers(B0)

        @pl.when(p < npair)
        def _():
            prefetch_idx(2 * p + 2, B0)

        @pl.when(p > 0)
        def _():
            wait_scatters(B0)

        compute_and_scatter(2 * p, B0)

        @pl.when(p < npair)
        def _():
            issue_gathers(2 * p + 2, B0)

        wait_gathers(B1)

        @pl.when(p < npair - 1)
        def _():
            prefetch_idx(2 * p + 3, B1)

        @pl.when(p > 0)
        def _():
            wait_scatters(B1)

        compute_and_scatter(2 * p + 1, B1)

        @pl.when(p < npair - 1)
        def _():
            issue_gathers(2 * p + 3, B1)

        return carry

    lax.fori_loop(0, npair, pair, 0)
    wait_gathers(B0)
    wait_scatters(B0)
    compute_and_scatter(BLK2 - 1, B0)
    wait_scatters(B0)
    wait_scatters(B1)
    plsc.subcore_barrier()
    orow = cid * NPAD + r0
    pltpu.sync_copy(accW.at[pl.ds(r0, RPT), :], outW.at[pl.ds(orow, RPT), :])


def _l3_buf_set():
    return [pltpu.VMEM((EB,), jnp.int32),
            pltpu.VMEM((EB,), jnp.int32),
            pltpu.VMEM((EB,), jnp.int32),
            pltpu.VMEM((EB, 16), _f32),
            pltpu.VMEM((EB, 16), _f32),
            pltpu.VMEM((EB, 16), _f32),
            pltpu.SemaphoreType.DMA,
            pltpu.SemaphoreType.DMA,
            pltpu.SemaphoreType.DMA,
            pltpu.SemaphoreType.DMA,
            pltpu.SemaphoreType.DMA,
            pltpu.SemaphoreType.DMA]


def _sc_l3_call(sF, dF, zas, ad3, z16):
    out_type = jax.ShapeDtypeStruct((2 * NPAD, 16), _f32)
    scratch = ([pltpu.VMEM_SHARED((NPAD, 16), _f32)]
               + _l3_buf_set() + _l3_buf_set())
    fn = pl.kernel(_sc_l3_body, out_type=out_type, mesh=_sc_mesh(),
                   scratch_types=scratch,
                   compiler_params=pltpu.CompilerParams(
                       use_tc_tiling_on_sc=False))
    return fn(sF, dF, zas, ad3, z16)


def _tc_a_body(x_ref, Wf_ref, Wu_ref, PsF_ref, PdF_ref, PsU_ref, PdU_ref,
               hsF_ref, hsU_ref, adF_ref, adU_ref):
    x = x_ref[...]
    hf = jnp.dot(x, Wf_ref[...], preferred_element_type=_f32)
    hu = jnp.dot(x, Wu_ref[...], preferred_element_type=_f32)
    asf = jnp.dot(hf, PsF_ref[...], preferred_element_type=_f32)
    asu = jnp.dot(hu, PsU_ref[...], preferred_element_type=_f32)
    hsF_ref[...] = jnp.concatenate([hf, asf], axis=1)
    hsU_ref[...] = jnp.concatenate([hu, asu], axis=1)
    adF_ref[...] = jnp.dot(hf, PdF_ref[...], preferred_element_type=_f32)
    adU_ref[...] = jnp.dot(hu, PdU_ref[...], preferred_element_type=_f32)


def _tc_a_call(x_pad, Wf, Wu, PsF, PdF, PsU, PdU):
    w_spec = pl.BlockSpec((IN_DIM, 64), lambda i: (0, 0))
    p_spec = pl.BlockSpec((64, 16), lambda i: (0, 0))
    row80 = pl.BlockSpec((512, 80), lambda i: (i, 0))
    row16 = pl.BlockSpec((512, 16), lambda i: (i, 0))
    return pl.pallas_call(
        _tc_a_body,
        grid=(ROWB,),
        in_specs=[pl.BlockSpec((512, IN_DIM), lambda i: (i, 0)),
                  w_spec, w_spec, p_spec, p_spec, p_spec, p_spec],
        out_specs=[row80, row80, row16, row16],
        out_shape=[jax.ShapeDtypeStruct((NPAD, 80), _f32),
                   jax.ShapeDtypeStruct((NPAD, 80), _f32),
                   jax.ShapeDtypeStruct((NPAD, 16), _f32),
                   jax.ShapeDtypeStruct((NPAD, 16), _f32)],
    )(x_pad, Wf, Wu, PsF, PdF, PsU, PdU)


def _tc_b_body(aHF_ref, aHU_ref, bf_ref, bu_ref,
               Wfc_ref, bfc_ref, wo_ref, aos_ref, aod_ref,
               zas_ref, ad3_ref):
    eps = 1e-16
    hf = aHF_ref[...]
    f0 = hf[:, 0:32] / (hf[:, 64:65] + eps)
    f1 = hf[:, 32:64] / (hf[:, 65:66] + eps)
    outF = jnp.maximum(jnp.concatenate([f0, f1], axis=1) + bf_ref[...], 0.0)
    hu = aHU_ref[...]
    u0 = hu[:, 0:32] / (hu[:, 64:65] + eps)
    u1 = hu[:, 32:64] / (hu[:, 65:66] + eps)
    outU = jnp.maximum(jnp.concatenate([u0, u1], axis=1) + bu_ref[...], 0.0)
    cat = jnp.concatenate([outF, outU], axis=1)
    h2 = jnp.dot(cat, Wfc_ref[...], preferred_element_type=_f32) + bfc_ref[...]
    h2 = jnp.maximum(h2, 0.0)
    z = jnp.sum(h2 * wo_ref[...], axis=1, keepdims=True)
    zeros14 = jnp.zeros((z.shape[0], 14), _f32)
    zeros15 = jnp.zeros((z.shape[0], 15), _f32)
    zas_ref[...] = jnp.concatenate([z * aos_ref[0, 0], z, zeros14], axis=1)
    ad3_ref[...] = jnp.concatenate([z * aod_ref[0, 0], zeros15], axis=1)


def _tc_b_call(aHF, aHU, bf, bu, Wfc, bfc, wo, aos, aod):
    row80 = pl.BlockSpec((512, 80), lambda i: (i, 0))
    row16 = pl.BlockSpec((512, 16), lambda i: (i, 0))
    b_spec = pl.BlockSpec((1, 64), lambda i: (0, 0))
    s_spec = pl.BlockSpec((1, 1), lambda i: (0, 0))
    return pl.pallas_call(
        _tc_b_body,
        grid=(ROWB,),
        in_specs=[row80, row80, b_spec, b_spec,
                  pl.BlockSpec((128, 64), lambda i: (0, 0)), b_spec,
                  b_spec, s_spec, s_spec],
        out_specs=[row16, row16],
        out_shape=[jax.ShapeDtypeStruct((NPAD, 16), _f32),
                   jax.ShapeDtypeStruct((NPAD, 16), _f32)],
    )(aHF, aHU, bf, bu, Wfc, bfc, wo, aos, aod)


def _tc_c_body(pW0_ref, pW1_ref, bo_ref, out_ref):
    den = pW0_ref[:, 0:1] + pW1_ref[:, 0:1]
    num = pW0_ref[:, 1:2] + pW1_ref[:, 1:2]
    out_ref[...] = jax.nn.sigmoid(num / (den + 1e-16) + bo_ref[0, 0])


def _tc_c_call(pW, bo2):
    lo = pl.BlockSpec((512, 16), lambda i: (i, 0))
    hi = pl.BlockSpec((512, 16), lambda i: (i + ROWB, 0))
    s_spec = pl.BlockSpec((1, 1), lambda i: (0, 0))
    return pl.pallas_call(
        _tc_c_body,
        grid=(ROWB,),
        in_specs=[lo, hi, s_spec],
        out_specs=pl.BlockSpec((512, 1), lambda i: (i, 0)),
        out_shape=jax.ShapeDtypeStruct((NPAD, 1), _f32),
    )(pW, pW, bo2)


def _alpha_proj(a):
    P = jnp.zeros((64, 16), _f32)
    P = P.at[0:32, 0].set(a[0])
    P = P.at[32:64, 1].set(a[1])
    return P


def kernel(x, edge_index, Wf, af_src, af_dst, bf, Wu, au_src, au_dst, bu,
           Wfc, bfc, Wo, ao_src, ao_dst, bo):
    src = edge_index[0]
    dst = edge_index[1]
    loop = jnp.arange(N, dtype=jnp.int32)
    padv = jnp.full((EPAD - E_TOT,), N, dtype=jnp.int32)
    sF = jnp.concatenate([src, loop, padv])
    dF = jnp.concatenate([dst, loop, padv])
    x_pad = jnp.zeros((NPAD, IN_DIM), _f32).at[:N].set(x)
    z80 = jnp.zeros((NPAD, 80), _f32)
    z16 = jnp.zeros((NPAD, 16), _f32)

    hsF, hsU, adF, adU = _tc_a_call(
        x_pad, Wf, Wu, _alpha_proj(af_src), _alpha_proj(af_dst),
        _alpha_proj(au_src), _alpha_proj(au_dst))

    outF, outU = _sc_fu_call(sF, dF, hsF, adF, hsU, adU, z80)

    zas, ad3 = _tc_b_call(outF, outU,
                          bf.reshape(1, 64), bu.reshape(1, 64),
                          Wfc, bfc.reshape(1, 64), Wo.reshape(1, 64),
                          ao_src.reshape(1, 1), ao_dst.reshape(1, 1))

    pW = _sc_l3_call(sF, dF, zas, ad3, z16)

    out = _tc_c_call(pW, bo.reshape(1, 1))
    return out[:N]


# speedup vs baseline: 1.4350x; 1.0038x over previous
import functools

import jax
import jax.numpy as jnp
from jax import lax
from jax.experimental import pallas as pl
from jax.experimental.pallas import tpu as pltpu
from jax.experimental.pallas import tpu_sc as plsc

N = 10000
IN_DIM = 128
HID = 64
NPAD = 10240
EB = 128
E_TOT = 320000 + N
EPAD = 331776
BLK1 = 162
BLK2 = 81
RPT = NPAD // 16
ROWB = NPAD // 512

_f32 = jnp.float32


def _sc_mesh():
    return plsc.VectorSubcoreMesh(core_axis_name="c", subcore_axis_name="s")


def _gat_edge_pass(sid, gidx_hbm, sidx_hbm, hs_tab, ad_tab,
                   accH, z80, outH, bufs):
    r0 = sid * RPT
    pltpu.sync_copy(z80.at[pl.ds(r0, RPT), :], accH.at[pl.ds(r0, RPT), :])
    plsc.subcore_barrier()

    def prefetch_idx(blk, B):
        gi, si, ssi, hr, br, ob, sg1, sg3, ss1, sx1, sx2, sx3 = B
        base = sid * (BLK1 * EB) + blk * EB
        pltpu.async_copy(gidx_hbm.at[pl.ds(base, EB)], gi, sx1)
        pltpu.async_copy(sidx_hbm.at[pl.ds(base, EB)], si, sx2)

    def issue_gathers(blk, B):
        gi, si, ssi, hr, br, ob, sg1, sg3, ss1, sx1, sx2, sx3 = B
        base = sid * (BLK1 * EB) + blk * EB
        pltpu.make_async_copy(gidx_hbm.at[pl.ds(base, EB)], gi, sx1).wait()
        pltpu.make_async_copy(sidx_hbm.at[pl.ds(base, EB)], si, sx2).wait()
        pltpu.async_copy(hs_tab.at[gi], hr, sg1)
        pltpu.async_copy(ad_tab.at[si], br, sg3)

    def wait_gathers(B):
        gi, si, ssi, hr, br, ob, sg1, sg3, ss1, sx1, sx2, sx3 = B
        pltpu.make_async_copy(hs_tab.at[gi], hr, sg1).wait()
        pltpu.make_async_copy(ad_tab.at[si], br, sg3).wait()

    def wait_scatters(B):
        gi, si, ssi, hr, br, ob, sg1, sg3, ss1, sx1, sx2, sx3 = B
        pltpu.make_async_copy(ob, accH.at[ssi], ss1).wait()

    def compute_and_scatter(blk, B):
        gi, si, ssi, hr, br, ob, sg1, sg3, ss1, sx1, sx2, sx3 = B
        base = sid * (BLK1 * EB) + blk * EB
        pltpu.async_copy(sidx_hbm.at[pl.ds(base, EB)], ssi, sx3)

        lane_zero = jnp.zeros((16,), jnp.int32)
        lane_one = jnp.ones((16,), jnp.int32)

        @plsc.parallel_loop(0, EB, unroll=16)
        def edge(e):
            v = hr[e, pl.ds(64, 16)] + br[e, :]
            w = jnp.exp(jnp.maximum(v, 0.2 * v))
            ob[e, pl.ds(64, 16)] = w
            w0 = jnp.take_along_axis(w, lane_zero, axis=0)
            w1 = jnp.take_along_axis(w, lane_one, axis=0)
            ob[e, pl.ds(0, 16)] = hr[e, pl.ds(0, 16)] * w0
            ob[e, pl.ds(16, 16)] = hr[e, pl.ds(16, 16)] * w0
            ob[e, pl.ds(32, 16)] = hr[e, pl.ds(32, 16)] * w1
            ob[e, pl.ds(48, 16)] = hr[e, pl.ds(48, 16)] * w1

        pltpu.make_async_copy(sidx_hbm.at[pl.ds(base, EB)], ssi, sx3).wait()
        pltpu.async_copy(ob, accH.at[ssi], ss1, add=True)

    ntri = BLK1 // 3
    for k, B in enumerate(bufs):
        prefetch_idx(k, B)
    for k, B in enumerate(bufs):
        issue_gathers(k, B)

    def triple(p, carry):
        for k, B in enumerate(bufs):
            wait_gathers(B)

            @pl.when(p < ntri - 1)
            def _():
                prefetch_idx(3 * p + 3 + k, B)

            @pl.when(p > 0)
            def _():
                wait_scatters(B)

            compute_and_scatter(3 * p + k, B)

            @pl.when(p < ntri - 1)
            def _():
                issue_gathers(3 * p + 3 + k, B)

        return carry

    lax.fori_loop(0, ntri, triple, 0)
    for B in bufs:
        wait_scatters(B)
    plsc.subcore_barrier()
    pltpu.sync_copy(accH.at[pl.ds(r0, RPT), :], outH.at[pl.ds(r0, RPT), :])


def _buf_set():
    return [pltpu.VMEM((EB,), jnp.int32),
            pltpu.VMEM((EB,), jnp.int32),
            pltpu.VMEM((EB,), jnp.int32),
            pltpu.VMEM((EB, 80), _f32),
            pltpu.VMEM((EB, 16), _f32),
            pltpu.VMEM((EB, 80), _f32),
            pltpu.SemaphoreType.DMA,
            pltpu.SemaphoreType.DMA,
            pltpu.SemaphoreType.DMA,
            pltpu.SemaphoreType.DMA,
            pltpu.SemaphoreType.DMA,
            pltpu.SemaphoreType.DMA]


def _sc_fu_body(sF, dF, hsF, adF, hsU, adU, z80,
                outF, outU, accH, *flat_bufs):
    cid = lax.axis_index("c")
    sid = lax.axis_index("s")
    nb = len(flat_bufs) // 3
    bufs = (flat_bufs[:nb], flat_bufs[nb:2 * nb], flat_bufs[2 * nb:])

    @pl.when(cid == 0)
    def _():
        _gat_edge_pass(sid, sF, dF, hsF, adF, accH, z80, outF, bufs)

    @pl.when(cid == 1)
    def _():
        _gat_edge_pass(sid, dF, sF, hsU, adU, accH, z80, outU, bufs)


def _sc_fu_call(sF, dF, hsF, adF, hsU, adU, z80):
    out_type = [jax.ShapeDtypeStruct((NPAD, 80), _f32),
                jax.ShapeDtypeStruct((NPAD, 80), _f32)]
    scratch = ([pltpu.VMEM_SHARED((NPAD, 80), _f32)]
               + _buf_set() + _buf_set() + _buf_set())
    fn = pl.kernel(_sc_fu_body, out_type=out_type, mesh=_sc_mesh(),
                   scratch_types=scratch,
                   compiler_params=pltpu.CompilerParams(
                       use_tc_tiling_on_sc=False))
    return fn(sF, dF, hsF, adF, hsU, adU, z80)


def _sc_l3_body(sF, dF, zas, ad3, z16, outW, accW, *flat_bufs):
    cid = lax.axis_index("c")
    sid = lax.axis_index("s")
    wid = cid * 16 + sid
    r0 = sid * RPT
    pltpu.sync_copy(z16.at[pl.ds(r0, RPT), :], accW.at[pl.ds(r0, RPT), :])
    plsc.subcore_barrier()
    lane0 = lax.iota(jnp.int32, 16) == 0
    lane1 = lax.iota(jnp.int32, 16) == 1
    nb = len(flat_bufs) // 3
    bufs3 = (flat_bufs[:nb], flat_bufs[nb:2 * nb], flat_bufs[2 * nb:])

    def prefetch_idx(blk, B):
        gi, si, ssi, zr, ar, wb, sg1, sg2, ss2, sx1, sx2, sx3 = B
        base = wid * (BLK2 * EB) + blk * EB
        pltpu.async_copy(sF.at[pl.ds(base, EB)], gi, sx1)
        pltpu.async_copy(dF.at[pl.ds(base, EB)], si, sx2)

    def issue_gathers(blk, B):
        gi, si, ssi, zr, ar, wb, sg1, sg2, ss2, sx1, sx2, sx3 = B
        base = wid * (BLK2 * EB) + blk * EB
        pltpu.make_async_copy(sF.at[pl.ds(base, EB)], gi, sx1).wait()
        pltpu.make_async_copy(dF.at[pl.ds(base, EB)], si, sx2).wait()
        pltpu.async_copy(zas.at[gi], zr, sg1)
        pltpu.async_copy(ad3.at[si], ar, sg2)

    def wait_gathers(B):
        gi, si, ssi, zr, ar, wb, sg1, sg2, ss2, sx1, sx2, sx3 = B
        pltpu.make_async_copy(zas.at[gi], zr, sg1).wait()
        pltpu.make_async_copy(ad3.at[si], ar, sg2).wait()

    def wait_scatters(B):
        gi, si, ssi, zr, ar, wb, sg1, sg2, ss2, sx1, sx2, sx3 = B
        pltpu.make_async_copy(wb, accW.at[ssi], ss2).wait()

    def compute_and_scatter(blk, B):
        gi, si, ssi, zr, ar, wb, sg1, sg2, ss2, sx1, sx2, sx3 = B
        base = wid * (BLK2 * EB) + blk * EB
        pltpu.async_copy(dF.at[pl.ds(base, EB)], ssi, sx3)

        lane_zero = jnp.zeros((16,), jnp.int32)
        lane_one = jnp.ones((16,), jnp.int32)

        @plsc.parallel_loop(0, EB, unroll=16)
        def edge(e):
            vz = zr[e, :]
            v = vz + ar[e, :]
            w = jnp.exp(jnp.maximum(v, 0.2 * v))
            w0 = jnp.take_along_axis(w, lane_zero, axis=0)
            zv = jnp.take_along_axis(vz, lane_one, axis=0)
            wb[e, :] = jnp.where(lane0, w0, jnp.where(lane1, w0 * zv, 0.0))

        pltpu.make_async_copy(dF.at[pl.ds(base, EB)], ssi, sx3).wait()
        pltpu.async_copy(wb, accW.at[ssi], ss2, add=True)

    ntri = BLK2 // 3
    for k, B in enumerate(bufs3):
        prefetch_idx(k, B)
    for k, B in enumerate(bufs3):
        issue_gathers(k, B)

    def triple(p, carry):
        for k, B in enumerate(bufs3):
            wait_gathers(B)

            @pl.when(p < ntri - 1)
            def _():
                prefetch_idx(3 * p + 3 + k, B)

            @pl.when(p > 0)
            def _():
                wait_scatters(B)

            compute_and_scatter(3 * p + k, B)

            @pl.when(p < ntri - 1)
            def _():
                issue_gathers(3 * p + 3 + k, B)

        return carry

    lax.fori_loop(0, ntri, triple, 0)
    for B in bufs3:
        wait_scatters(B)
    plsc.subcore_barrier()
    orow = cid * NPAD + r0
    pltpu.sync_copy(accW.at[pl.ds(r0, RPT), :], outW.at[pl.ds(orow, RPT), :])


def _l3_buf_set():
    return [pltpu.VMEM((EB,), jnp.int32),
            pltpu.VMEM((EB,), jnp.int32),
            pltpu.VMEM((EB,), jnp.int32),
            pltpu.VMEM((EB, 16), _f32),
            pltpu.VMEM((EB, 16), _f32),
            pltpu.VMEM((EB, 16), _f32),
            pltpu.SemaphoreType.DMA,
            pltpu.SemaphoreType.DMA,
            pltpu.SemaphoreType.DMA,
            pltpu.SemaphoreType.DMA,
            pltpu.SemaphoreType.DMA,
            pltpu.SemaphoreType.DMA]


def _sc_l3_call(sF, dF, zas, ad3, z16):
    out_type = jax.ShapeDtypeStruct((2 * NPAD, 16), _f32)
    scratch = ([pltpu.VMEM_SHARED((NPAD, 16), _f32)]
               + _l3_buf_set() + _l3_buf_set() + _l3_buf_set())
    fn = pl.kernel(_sc_l3_body, out_type=out_type, mesh=_sc_mesh(),
                   scratch_types=scratch,
                   compiler_params=pltpu.CompilerParams(
                       use_tc_tiling_on_sc=False))
    return fn(sF, dF, zas, ad3, z16)


def _tc_a_body(x_ref, Wf_ref, Wu_ref, PsF_ref, PdF_ref, PsU_ref, PdU_ref,
               hsF_ref, hsU_ref, adF_ref, adU_ref):
    x = x_ref[...]
    hf = jnp.dot(x, Wf_ref[...], preferred_element_type=_f32)
    hu = jnp.dot(x, Wu_ref[...], preferred_element_type=_f32)
    asf = jnp.dot(hf, PsF_ref[...], preferred_element_type=_f32)
    asu = jnp.dot(hu, PsU_ref[...], preferred_element_type=_f32)
    hsF_ref[...] = jnp.concatenate([hf, asf], axis=1)
    hsU_ref[...] = jnp.concatenate([hu, asu], axis=1)
    adF_ref[...] = jnp.dot(hf, PdF_ref[...], preferred_element_type=_f32)
    adU_ref[...] = jnp.dot(hu, PdU_ref[...], preferred_element_type=_f32)


def _tc_a_call(x_pad, Wf, Wu, PsF, PdF, PsU, PdU):
    w_spec = pl.BlockSpec((IN_DIM, 64), lambda i: (0, 0))
    p_spec = pl.BlockSpec((64, 16), lambda i: (0, 0))
    row80 = pl.BlockSpec((512, 80), lambda i: (i, 0))
    row16 = pl.BlockSpec((512, 16), lambda i: (i, 0))
    return pl.pallas_call(
        _tc_a_body,
        grid=(ROWB,),
        in_specs=[pl.BlockSpec((512, IN_DIM), lambda i: (i, 0)),
                  w_spec, w_spec, p_spec, p_spec, p_spec, p_spec],
        out_specs=[row80, row80, row16, row16],
        out_shape=[jax.ShapeDtypeStruct((NPAD, 80), _f32),
                   jax.ShapeDtypeStruct((NPAD, 80), _f32),
                   jax.ShapeDtypeStruct((NPAD, 16), _f32),
                   jax.ShapeDtypeStruct((NPAD, 16), _f32)],
    )(x_pad, Wf, Wu, PsF, PdF, PsU, PdU)


def _tc_b_body(aHF_ref, aHU_ref, bf_ref, bu_ref,
               Wfc_ref, bfc_ref, wo_ref, aos_ref, aod_ref,
               zas_ref, ad3_ref):
    eps = 1e-16
    hf = aHF_ref[...]
    f0 = hf[:, 0:32] / (hf[:, 64:65] + eps)
    f1 = hf[:, 32:64] / (hf[:, 65:66] + eps)
    outF = jnp.maximum(jnp.concatenate([f0, f1], axis=1) + bf_ref[...], 0.0)
    hu = aHU_ref[...]
    u0 = hu[:, 0:32] / (hu[:, 64:65] + eps)
    u1 = hu[:, 32:64] / (hu[:, 65:66] + eps)
    outU = jnp.maximum(jnp.concatenate([u0, u1], axis=1) + bu_ref[...], 0.0)
    cat = jnp.concatenate([outF, outU], axis=1)
    h2 = jnp.dot(cat, Wfc_ref[...], preferred_element_type=_f32) + bfc_ref[...]
    h2 = jnp.maximum(h2, 0.0)
    z = jnp.sum(h2 * wo_ref[...], axis=1, keepdims=True)
    zeros14 = jnp.zeros((z.shape[0], 14), _f32)
    zeros15 = jnp.zeros((z.shape[0], 15), _f32)
    zas_ref[...] = jnp.concatenate([z * aos_ref[0, 0], z, zeros14], axis=1)
    ad3_ref[...] = jnp.concatenate([z * aod_ref[0, 0], zeros15], axis=1)


def _tc_b_call(aHF, aHU, bf, bu, Wfc, bfc, wo, aos, aod):
    row80 = pl.BlockSpec((512, 80), lambda i: (i, 0))
    row16 = pl.BlockSpec((512, 16), lambda i: (i, 0))
    b_spec = pl.BlockSpec((1, 64), lambda i: (0, 0))
    s_spec = pl.BlockSpec((1, 1), lambda i: (0, 0))
    return pl.pallas_call(
        _tc_b_body,
        grid=(ROWB,),
        in_specs=[row80, row80, b_spec, b_spec,
                  pl.BlockSpec((128, 64), lambda i: (0, 0)), b_spec,
                  b_spec, s_spec, s_spec],
        out_specs=[row16, row16],
        out_shape=[jax.ShapeDtypeStruct((NPAD, 16), _f32),
                   jax.ShapeDtypeStruct((NPAD, 16), _f32)],
    )(aHF, aHU, bf, bu, Wfc, bfc, wo, aos, aod)


def _tc_c_body(pW0_ref, pW1_ref, bo_ref, out_ref):
    den = pW0_ref[:, 0:1] + pW1_ref[:, 0:1]
    num = pW0_ref[:, 1:2] + pW1_ref[:, 1:2]
    out_ref[...] = jax.nn.sigmoid(num / (den + 1e-16) + bo_ref[0, 0])


def _tc_c_call(pW, bo2):
    lo = pl.BlockSpec((512, 16), lambda i: (i, 0))
    hi = pl.BlockSpec((512, 16), lambda i: (i + ROWB, 0))
    s_spec = pl.BlockSpec((1, 1), lambda i: (0, 0))
    return pl.pallas_call(
        _tc_c_body,
        grid=(ROWB,),
        in_specs=[lo, hi, s_spec],
        out_specs=pl.BlockSpec((512, 1), lambda i: (i, 0)),
        out_shape=jax.ShapeDtypeStruct((NPAD, 1), _f32),
    )(pW, pW, bo2)


def _alpha_proj(a):
    P = jnp.zeros((64, 16), _f32)
    P = P.at[0:32, 0].set(a[0])
    P = P.at[32:64, 1].set(a[1])
    return P


def kernel(x, edge_index, Wf, af_src, af_dst, bf, Wu, au_src, au_dst, bu,
           Wfc, bfc, Wo, ao_src, ao_dst, bo):
    src = edge_index[0]
    dst = edge_index[1]
    loop = jnp.arange(N, dtype=jnp.int32)
    padv = jnp.full((EPAD - E_TOT,), N, dtype=jnp.int32)
    sF = jnp.concatenate([src, loop, padv])
    dF = jnp.concatenate([dst, loop, padv])
    x_pad = jnp.zeros((NPAD, IN_DIM), _f32).at[:N].set(x)
    z80 = jnp.zeros((NPAD, 80), _f32)
    z16 = jnp.zeros((NPAD, 16), _f32)

    hsF, hsU, adF, adU = _tc_a_call(
        x_pad, Wf, Wu, _alpha_proj(af_src), _alpha_proj(af_dst),
        _alpha_proj(au_src), _alpha_proj(au_dst))

    outF, outU = _sc_fu_call(sF, dF, hsF, adF, hsU, adU, z80)

    zas, ad3 = _tc_b_call(outF, outU,
                          bf.reshape(1, 64), bu.reshape(1, 64),
                          Wfc, bfc.reshape(1, 64), Wo.reshape(1, 64),
                          ao_src.reshape(1, 1), ao_dst.reshape(1, 1))

    pW = _sc_l3_call(sF, dF, zas, ad3, z16)

    out = _tc_c_call(pW, bo.reshape(1, 1))
    return out[:N]
